# baseline (speedup 1.0000x reference)
import functools

import jax
import jax.numpy as jnp
from jax import lax
from jax.experimental import pallas as pl
from jax.experimental.pallas import tpu as pltpu
from jax.experimental.pallas import tpu_sc as plsc

C = 9
TOTAL = 16 * 2048
NC, NS, L = 2, 16, 16
NW = NC * NS
RPW = TOTAL // NW
NG = RPW // L

_LN2 = 0.6931471805599453
_SQRT2 = 1.4142135623730951


def _ln(s):
    bits = lax.bitcast_convert_type(s, jnp.int32)
    e = lax.shift_right_arithmetic(bits, 23) - 127
    m = lax.bitcast_convert_type((bits & 0x007FFFFF) | 0x3F800000, jnp.float32)
    big = m > _SQRT2
    m = jnp.where(big, m * 0.5, m)
    e = e + jnp.where(big, 1, 0)
    t = (m - 1.0) / (m + 1.0)
    t2 = t * t
    p = 1.0 / 9.0
    p = p * t2 + 1.0 / 7.0
    p = p * t2 + 1.0 / 5.0
    p = p * t2 + 1.0 / 3.0
    p = p * t2 + 1.0
    return e.astype(jnp.float32) * _LN2 + 2.0 * t * p


@functools.partial(
    pl.kernel,
    out_type=jax.ShapeDtypeStruct((NW, L), jnp.float32),
    mesh=plsc.VectorSubcoreMesh(
        core_axis_name="c", subcore_axis_name="s", num_cores=NC, num_subcores=NS
    ),
    scratch_types=[
        pltpu.VMEM((C, RPW), jnp.float32),
        pltpu.VMEM((RPW,), jnp.int32),
        pltpu.VMEM((RPW,), jnp.float32),
        pltpu.VMEM((C * L,), jnp.float32),
        pltpu.VMEM((L,), jnp.float32),
        pltpu.SemaphoreType.DMA,
    ],
)
def _focal_partials(x_hbm, tgt_hbm, msk_hbm, alpha_hbm, out_hbm,
                    x_v, tgt_v, msk_v, alpha_v, out_v, dma_sem):
    wid = lax.axis_index("s") * NC + lax.axis_index("c")
    copies = [
        pltpu.async_copy(x_hbm.at[:, pl.ds(wid * RPW, RPW)], x_v, dma_sem),
        pltpu.async_copy(tgt_hbm.at[pl.ds(wid * RPW, RPW)], tgt_v, dma_sem),
        pltpu.async_copy(msk_hbm.at[pl.ds(wid * RPW, RPW)], msk_v, dma_sem),
        pltpu.async_copy(alpha_hbm, alpha_v, dma_sem),
    ]
    for cp in copies:
        cp.wait()

    lanes = lax.iota(jnp.int32, L)
    av = [alpha_v[pl.ds(c * L, L)] for c in range(C)]

    def body(g, carry):
        num, den = carry
        off = g * L
        tgt = tgt_v[pl.ds(off, L)]
        w = jnp.where(msk_v[pl.ds(off, L)] == 1.0, 1.0, 0.0)
        vs = [x_v[c, pl.ds(off, L)] for c in range(C)]
        mx = vs[0]
        for v in vs[1:]:
            mx = jnp.maximum(mx, v)
        es = [jnp.exp(v - mx) for v in vs]
        s = es[0]
        for e in es[1:]:
            s = s + e
        zero = jnp.zeros((L,), jnp.float32)
        ptn, x_t, a_t = zero, zero, zero
        for c in range(C):
            hit = tgt == c
            ptn = ptn + jnp.where(hit, es[c], 0.0)
            x_t = x_t + jnp.where(hit, vs[c], 0.0)
            a_t = a_t + jnp.where(hit, av[c], 0.0)
        pt = ptn / s
        log_pt = (x_t - mx) - _ln(s)
        om = 1.0 - pt
        return num + a_t * om * om * (-log_pt) * w, den + w

    zero = jnp.zeros((L,), jnp.float32)
    num, den = plsc.parallel_loop(0, NG, step=1, unroll=4, carry=(zero, zero))(
        lambda g, carry: body(g, carry)
    )
    num_s, den_s = num[0], den[0]
    for i in range(1, L):
        num_s = num_s + num[i]
        den_s = den_s + den[i]
    out_v[...] = jnp.where(lanes == 0, num_s, jnp.where(lanes == 1, den_s, 0.0))
    pltpu.sync_copy(out_v, out_hbm.at[wid])


def kernel(inputs, attention_mask, targets, alpha):
    x = (inputs.transpose(2, 0, 1)
         .reshape(C, 2, 8, 16, 128)
         .transpose(0, 1, 3, 2, 4)
         .reshape(C, TOTAL))
    tgt = (targets.reshape(2, 8, 16, 128).transpose(0, 2, 1, 3)
           .reshape(-1).astype(jnp.int32))
    msk = (attention_mask.reshape(2, 8, 16, 128).transpose(0, 2, 1, 3)
           .reshape(-1))
    a_rep = jnp.broadcast_to(alpha.reshape(C, 1), (C, L)).reshape(-1)
    parts = _focal_partials(x, tgt, msk, a_rep)
    return parts[:, 0].sum() / parts[:, 1].sum()


# speedup vs baseline: 1.0769x; 1.0769x over previous
import functools

import jax
import jax.numpy as jnp
from jax import lax
from jax.experimental import pallas as pl
from jax.experimental.pallas import tpu as pltpu
from jax.experimental.pallas import tpu_sc as plsc

C = 9
TOTAL = 16 * 2048
NC, NS, L = 2, 16, 16
NW = NC * NS
RPW = TOTAL // NW
NG = RPW // L

_LN2 = 0.6931471805599453
_SQRT2 = 1.4142135623730951


def _ln(s):
    bits = lax.bitcast_convert_type(s, jnp.int32)
    e = lax.shift_right_arithmetic(bits, 23) - 127
    m = lax.bitcast_convert_type((bits & 0x007FFFFF) | 0x3F800000, jnp.float32)
    big = m > _SQRT2
    m = jnp.where(big, m * 0.5, m)
    e = e + jnp.where(big, 1, 0)
    t = (m - 1.0) / (m + 1.0)
    t2 = t * t
    p = 1.0 / 9.0
    p = p * t2 + 1.0 / 7.0
    p = p * t2 + 1.0 / 5.0
    p = p * t2 + 1.0 / 3.0
    p = p * t2 + 1.0
    return e.astype(jnp.float32) * _LN2 + 2.0 * t * p


@functools.partial(
    pl.kernel,
    out_type=jax.ShapeDtypeStruct((NW, L), jnp.float32),
    mesh=plsc.VectorSubcoreMesh(
        core_axis_name="c", subcore_axis_name="s", num_cores=NC, num_subcores=NS
    ),
    scratch_types=[
        pltpu.VMEM((RPW * C,), jnp.float32),
        pltpu.VMEM((RPW,), jnp.int32),
        pltpu.VMEM((RPW,), jnp.float32),
        pltpu.VMEM((C * L,), jnp.float32),
        pltpu.VMEM((L,), jnp.float32),
        pltpu.SemaphoreType.DMA,
    ],
)
def _focal_partials(x_hbm, tgt_hbm, msk_hbm, alpha_hbm, out_hbm,
                    x_v, tgt_v, msk_v, alpha_v, out_v, dma_sem):
    wid = lax.axis_index("s") * NC + lax.axis_index("c")
    copies = [
        pltpu.async_copy(
            x_hbm.at[pl.ds((c * NW + wid) * RPW, RPW)],
            x_v.at[pl.ds(c * RPW, RPW)],
            dma_sem,
        )
        for c in range(C)
    ] + [
        pltpu.async_copy(tgt_hbm.at[pl.ds(wid * RPW, RPW)], tgt_v, dma_sem),
        pltpu.async_copy(msk_hbm.at[pl.ds(wid * RPW, RPW)], msk_v, dma_sem),
        pltpu.async_copy(alpha_hbm, alpha_v, dma_sem),
    ]
    for cp in copies:
        cp.wait()

    lanes = lax.iota(jnp.int32, L)
    av = [alpha_v[pl.ds(c * L, L)] for c in range(C)]

    def body(g, carry):
        num, den = carry
        off = g * L
        tgt = tgt_v[pl.ds(off, L)]
        w = jnp.where(msk_v[pl.ds(off, L)] == 1.0, 1.0, 0.0)
        vs = [x_v[pl.ds(c * RPW + off, L)] for c in range(C)]
        mx = vs[0]
        for v in vs[1:]:
            mx = jnp.maximum(mx, v)
        es = [jnp.exp(v - mx) for v in vs]
        s = es[0]
        for e in es[1:]:
            s = s + e
        zero = jnp.zeros((L,), jnp.float32)
        ptn, x_t, a_t = zero, zero, zero
        for c in range(C):
            hit = tgt == c
            ptn = ptn + jnp.where(hit, es[c], 0.0)
            x_t = x_t + jnp.where(hit, vs[c], 0.0)
            a_t = a_t + jnp.where(hit, av[c], 0.0)
        pt = ptn / s
        log_pt = (x_t - mx) - _ln(s)
        om = 1.0 - pt
        return num + a_t * om * om * (-log_pt) * w, den + w

    zero = jnp.zeros((L,), jnp.float32)
    num, den = plsc.parallel_loop(0, NG, step=1, unroll=4, carry=(zero, zero))(
        lambda g, carry: body(g, carry)
    )
    num_s, den_s = num[0], den[0]
    for i in range(1, L):
        num_s = num_s + num[i]
        den_s = den_s + den[i]
    out_v[...] = jnp.where(lanes == 0, num_s, jnp.where(lanes == 1, den_s, 0.0))
    pltpu.sync_copy(out_v, out_hbm.at[wid])


def kernel(inputs, attention_mask, targets, alpha):
    x = (inputs.transpose(2, 0, 1)
         .reshape(C, 2, 8, 16, 128)
         .transpose(0, 1, 3, 2, 4)
         .reshape(-1))
    tgt = (targets.reshape(2, 8, 16, 128).transpose(0, 2, 1, 3)
           .reshape(-1).astype(jnp.int32))
    msk = (attention_mask.reshape(2, 8, 16, 128).transpose(0, 2, 1, 3)
           .reshape(-1))
    a_rep = jnp.broadcast_to(alpha.reshape(C, 1), (C, L)).reshape(-1)
    parts = _focal_partials(x, tgt, msk, a_rep)
    return parts[:, 0].sum() / parts[:, 1].sum()


# speedup vs baseline: 1.1170x; 1.0372x over previous
import functools

import jax
import jax.numpy as jnp
from jax import lax
from jax.experimental import pallas as pl
from jax.experimental.pallas import tpu as pltpu
from jax.experimental.pallas import tpu_sc as plsc

C = 9
TOTAL = 16 * 2048
NC, NS, L = 2, 16, 16
NW = NC * NS
RPW = TOTAL // NW
NG = RPW // L

_LN2 = 0.6931471805599453
_LN_POLY = (
    -0.006151545067440741, 0.01024394858737874, -0.014338309622245023,
    0.02596728452889429, -0.049409622681556215, 0.09879175632379983,
    -0.22222136804592243, 0.6666661659415327, 0.4054651037918593,
)


def _ln(s):
    bits = lax.bitcast_convert_type(s, jnp.int32)
    e = lax.shift_right_arithmetic(bits, 23) - 127
    m = lax.bitcast_convert_type((bits & 0x007FFFFF) | 0x3F800000, jnp.float32)
    u = m - 1.5
    p = _LN_POLY[0]
    for c in _LN_POLY[1:]:
        p = p * u + c
    return e.astype(jnp.float32) * _LN2 + p


@functools.partial(
    pl.kernel,
    out_type=jax.ShapeDtypeStruct((NW, L), jnp.float32),
    mesh=plsc.VectorSubcoreMesh(
        core_axis_name="c", subcore_axis_name="s", num_cores=NC, num_subcores=NS
    ),
    scratch_types=[
        pltpu.VMEM((RPW * C,), jnp.float32),
        pltpu.VMEM((RPW,), jnp.int32),
        pltpu.VMEM((RPW,), jnp.float32),
        pltpu.VMEM((L,), jnp.float32),
        pltpu.VMEM((L,), jnp.float32),
        pltpu.SemaphoreType.DMA,
    ],
)
def _focal_partials(x_hbm, tgt_hbm, msk_hbm, alpha_hbm, out_hbm,
                    x_v, tgt_v, msk_v, alpha_v, out_v, dma_sem):
    wid = lax.axis_index("s") * NC + lax.axis_index("c")
    copies = [
        pltpu.async_copy(
            x_hbm.at[pl.ds((c * NW + wid) * RPW, RPW)],
            x_v.at[pl.ds(c * RPW, RPW)],
            dma_sem,
        )
        for c in range(C)
    ] + [
        pltpu.async_copy(tgt_hbm.at[pl.ds(wid * RPW, RPW)], tgt_v, dma_sem),
        pltpu.async_copy(msk_hbm.at[pl.ds(wid * RPW, RPW)], msk_v, dma_sem),
        pltpu.async_copy(alpha_hbm, alpha_v.at[pl.ds(0, C)], dma_sem),
    ]
    for cp in copies:
        cp.wait()

    lanes = lax.iota(jnp.int32, L)
    va = alpha_v[pl.ds(0, L)]
    av = [jnp.broadcast_to(va[c], (L,)) for c in range(C)]

    def body(g, carry):
        num, den = carry
        off = g * L
        tgt = tgt_v[pl.ds(off, L)]
        w = jnp.where(msk_v[pl.ds(off, L)] == 1.0, 1.0, 0.0)
        vs = [x_v[pl.ds(c * RPW + off, L)] for c in range(C)]
        mx = vs[0]
        for v in vs[1:]:
            mx = jnp.maximum(mx, v)
        es = [jnp.exp(v - mx) for v in vs]
        s = es[0]
        for e in es[1:]:
            s = s + e
        zero = jnp.zeros((L,), jnp.float32)
        x_t, a_t = zero, zero
        for c in range(C):
            hit = tgt == c
            x_t = x_t + jnp.where(hit, vs[c], 0.0)
            a_t = a_t + jnp.where(hit, av[c], 0.0)
        d = x_t - mx
        om = (s - jnp.exp(d)) * (1.0 / s)
        return num + a_t * om * om * (_ln(s) - d) * w, den + w

    zero = jnp.zeros((L,), jnp.float32)
    num, den = plsc.parallel_loop(0, NG, step=1, unroll=8, carry=(zero, zero))(
        lambda g, carry: body(g, carry)
    )
    num_s, den_s = num[0], den[0]
    for i in range(1, L):
        num_s = num_s + num[i]
        den_s = den_s + den[i]
    out_v[...] = jnp.where(lanes == 0, num_s, jnp.where(lanes == 1, den_s, 0.0))
    pltpu.sync_copy(out_v, out_hbm.at[wid])


def kernel(inputs, attention_mask, targets, alpha):
    x = (inputs.transpose(2, 0, 1)
         .reshape(C, 2, 8, 16, 128)
         .transpose(0, 1, 3, 2, 4)
         .reshape(-1))
    tgt = (targets.reshape(2, 8, 16, 128).transpose(0, 2, 1, 3)
           .reshape(-1).astype(jnp.int32))
    msk = (attention_mask.reshape(2, 8, 16, 128).transpose(0, 2, 1, 3)
           .reshape(-1))
    parts = _focal_partials(x, tgt, msk, alpha.reshape(C))
    return parts[:, 0].sum() / parts[:, 1].sum()
